# Initial kernel scaffold; baseline (speedup 1.0000x reference)
import jax
import jax.numpy as jnp
from jax.experimental import pallas as pl


def kernel(x_cat, x_num, tables, W1, b1, W2, b2):
    raise NotImplementedError("write your pallas kernel here")


# speedup vs baseline: 23.7080x; 23.7080x over previous
import functools

import jax
import jax.numpy as jnp
from jax import lax
from jax.experimental import pallas as pl
from jax.experimental.pallas import tpu as pltpu
from jax.experimental.pallas import tpu_sc as plsc

B, F, V, D, NUM, H, OUT = 16384, 26, 1000, 50, 64, 128, 2

def _tw_body(t_ref, w_ref, tw_ref):
    tw_ref[0] = jnp.dot(t_ref[0], w_ref[0], preferred_element_type=jnp.float32)


def _make_tw(tables, w1e):
    return pl.pallas_call(
        _tw_body,
        grid=(F,),
        in_specs=[
            pl.BlockSpec((1, V, D), lambda f: (f, 0, 0)),
            pl.BlockSpec((1, D, H), lambda f: (f, 0, 0)),
        ],
        out_specs=pl.BlockSpec((1, V, H), lambda f: (f, 0, 0)),
        out_shape=jax.ShapeDtypeStruct((F, V, H), jnp.float32),
    )(tables, w1e)


_ZBLK = 2048


def _z_body(x_ref, w_ref, b_ref, z_ref):
    z_ref[...] = (
        jnp.dot(x_ref[...], w_ref[...], preferred_element_type=jnp.float32)
        + b_ref[...]
    )


def _make_z(x_num, w1n, b1):
    return pl.pallas_call(
        _z_body,
        grid=(B // _ZBLK,),
        in_specs=[
            pl.BlockSpec((_ZBLK, NUM), lambda i: (i, 0)),
            pl.BlockSpec((NUM, H), lambda i: (0, 0)),
            pl.BlockSpec((1, H), lambda i: (0, 0)),
        ],
        out_specs=pl.BlockSpec((_ZBLK, H), lambda i: (i, 0)),
        out_shape=jax.ShapeDtypeStruct((B, H), jnp.float32),
    )(x_num, w1n, b1.reshape(1, H))


NC, NS = 2, 16
NW = NC * NS
BPW = B // NW
C = 16
IPC = C * F
CHUNKS = BPW // C
_SPLITS = (128, 128, 128, 32)


def _sc_body(idx_hbm, tw_hbm, z_hbm, out_hbm, idx_v, rows_v, acc_v, sem):
    wid = lax.axis_index("s") * NC + lax.axis_index("c")
    base_b = wid * BPW

    def chunk_body(ci, carry):
        b0 = base_b + ci * C
        i0 = b0 * F
        pltpu.sync_copy(idx_hbm.at[pl.ds(i0, IPC)], idx_v)
        copies = []
        off = 0
        for n in _SPLITS:
            copies.append(
                pltpu.async_copy(
                    tw_hbm.at[idx_v.at[pl.ds(off, n)]],
                    rows_v.at[pl.ds(off, n)],
                    sem,
                )
            )
            off += n
        pltpu.sync_copy(z_hbm.at[pl.ds(b0, C)], acc_v)
        for cp in copies:
            cp.wait()

        def b_body(bi, carry2):
            r0 = bi * F
            for h in range(H // 16):
                hs = pl.ds(h * 16, 16)
                s = rows_v[r0, hs]
                for f in range(1, F):
                    s = s + rows_v[r0 + f, hs]
                plsc.addupdate(acc_v.at[bi, hs], s)
            return carry2

        lax.fori_loop(0, C, b_body, 0, unroll=False)
        pltpu.sync_copy(acc_v, out_hbm.at[pl.ds(b0, C)])
        return carry

    lax.fori_loop(0, CHUNKS, chunk_body, 0, unroll=False)


def _make_hpre(flat_idx, tw_flat, z):
    mesh = plsc.VectorSubcoreMesh(core_axis_name="c", subcore_axis_name="s")
    f = functools.partial(
        pl.kernel,
        _sc_body,
        mesh=mesh,
        out_type=jax.ShapeDtypeStruct((B, H), jnp.float32),
        scratch_types=[
            pltpu.VMEM((IPC,), jnp.int32),
            pltpu.VMEM((IPC, H), jnp.float32),
            pltpu.VMEM((C, H), jnp.float32),
            pltpu.SemaphoreType.DMA,
        ],
    )()
    return f(flat_idx, tw_flat, z)


_OBLK = 2048
_OPAD = 8


def _out_body(h_ref, w2_ref, b2_ref, o_ref):
    hrelu = jnp.maximum(h_ref[...], 0.0)
    o_ref[...] = (
        jnp.dot(hrelu, w2_ref[...], preferred_element_type=jnp.float32)
        + b2_ref[...]
    )


def _make_out(hpre, w2p, b2p):
    return pl.pallas_call(
        _out_body,
        grid=(B // _OBLK,),
        in_specs=[
            pl.BlockSpec((_OBLK, H), lambda i: (i, 0)),
            pl.BlockSpec((H, _OPAD), lambda i: (0, 0)),
            pl.BlockSpec((1, _OPAD), lambda i: (0, 0)),
        ],
        out_specs=pl.BlockSpec((_OBLK, _OPAD), lambda i: (i, 0)),
        out_shape=jax.ShapeDtypeStruct((B, _OPAD), jnp.float32),
    )(hpre, w2p, b2p)


def kernel(x_cat, x_num, tables, W1, b1, W2, b2):
    w1e = W1[: F * D].reshape(F, D, H)
    w1n = W1[F * D :]
    tw = _make_tw(tables, w1e)
    z = _make_z(x_num, w1n, b1)

    flat_idx = (x_cat.astype(jnp.int32) + jnp.arange(F, dtype=jnp.int32) * V).reshape(-1)
    hpre = _make_hpre(flat_idx, tw.reshape(F * V, H), z)

    w2p = jnp.zeros((H, _OPAD), jnp.float32).at[:, :OUT].set(W2)
    b2p = jnp.zeros((1, _OPAD), jnp.float32).at[0, :OUT].set(b2)
    out = _make_out(hpre, w2p, b2p)
    return out[:, :OUT]


# speedup vs baseline: 35.0705x; 1.4793x over previous
import functools

import jax
import jax.numpy as jnp
from jax import lax
from jax.experimental import pallas as pl
from jax.experimental.pallas import tpu as pltpu
from jax.experimental.pallas import tpu_sc as plsc

B, F, V, D, NUM, H, OUT = 16384, 26, 1000, 50, 64, 128, 2

def _tw_body(t_ref, w_ref, tw_ref):
    tw_ref[0] = jnp.dot(t_ref[0], w_ref[0], preferred_element_type=jnp.float32)


def _make_tw(tables, w1e):
    return pl.pallas_call(
        _tw_body,
        grid=(F,),
        in_specs=[
            pl.BlockSpec((1, V, D), lambda f: (f, 0, 0)),
            pl.BlockSpec((1, D, H), lambda f: (f, 0, 0)),
        ],
        out_specs=pl.BlockSpec((1, V, H), lambda f: (f, 0, 0)),
        out_shape=jax.ShapeDtypeStruct((F, V, H), jnp.float32),
    )(tables, w1e)


NC, NS = 2, 16
NW = NC * NS
BPW = B // NW
C = 16
IPC = C * F
CHUNKS = BPW // C
_SPLITS = (128, 128, 128, 32)


def _sc_body(idx_hbm, tw_hbm, out_hbm, idx_v, rows0, rows1, acc0, acc1,
             gsem0, gsem1, osem0, osem1):
    wid = lax.axis_index("s") * NC + lax.axis_index("c")
    base_b = wid * BPW
    pltpu.sync_copy(idx_hbm.at[pl.ds(base_b * F, BPW * F)], idx_v)

    rows = (rows0, rows1)
    acc = (acc0, acc1)
    gsem = (gsem0, gsem1)
    osem = (osem0, osem1)

    def fire(ci, buf):
        off = 0
        for n in _SPLITS:
            pltpu.async_copy(
                tw_hbm.at[idx_v.at[pl.ds(ci * IPC + off, n)]],
                rows[buf].at[pl.ds(off, n)],
                gsem[buf],
            )
            off += n

    def drain_g(buf):
        pltpu.make_async_copy(tw_hbm.at[pl.ds(0, IPC)], rows[buf], gsem[buf]).wait()

    def drain_o(buf):
        pltpu.make_async_copy(acc[buf], out_hbm.at[pl.ds(0, C)], osem[buf]).wait()

    def compute(ci, buf):
        def b_body(bi, carry2):
            r0 = bi * F
            for h in range(H // 16):
                hs = pl.ds(h * 16, 16)
                s = rows[buf][r0, hs]
                for f in range(1, F):
                    s = s + rows[buf][r0 + f, hs]
                acc[buf][bi, hs] = s
            return carry2

        lax.fori_loop(0, C, b_body, 0, unroll=False)
        pltpu.async_copy(acc[buf], out_hbm.at[pl.ds(base_b + ci * C, C)], osem[buf])

    fire(0, 0)
    K = CHUNKS // 2

    def body(k, carry):
        a = 2 * k
        fire(a + 1, 1)
        drain_g(0)

        @pl.when(k > 0)
        def _():
            drain_o(0)

        compute(a, 0)

        @pl.when(k < K - 1)
        def _():
            fire(a + 2, 0)

        drain_g(1)

        @pl.when(k > 0)
        def _():
            drain_o(1)

        compute(a + 1, 1)
        return carry

    lax.fori_loop(0, K, body, 0, unroll=False)
    drain_o(0)
    drain_o(1)


def _make_gsum(flat_idx, tw_flat):
    mesh = plsc.VectorSubcoreMesh(core_axis_name="c", subcore_axis_name="s")
    f = functools.partial(
        pl.kernel,
        _sc_body,
        mesh=mesh,
        out_type=jax.ShapeDtypeStruct((B, H), jnp.float32),
        scratch_types=[
            pltpu.VMEM((BPW * F,), jnp.int32),
            pltpu.VMEM((IPC, H), jnp.float32),
            pltpu.VMEM((IPC, H), jnp.float32),
            pltpu.VMEM((C, H), jnp.float32),
            pltpu.VMEM((C, H), jnp.float32),
            pltpu.SemaphoreType.DMA,
            pltpu.SemaphoreType.DMA,
            pltpu.SemaphoreType.DMA,
            pltpu.SemaphoreType.DMA,
        ],
    )()
    return f(flat_idx, tw_flat)


_OBLK = 2048
_OPAD = 8


def _out_body(h_ref, x_ref, w1n_ref, b1_ref, w2_ref, b2_ref, o_ref):
    z = (
        jnp.dot(x_ref[...], w1n_ref[...], preferred_element_type=jnp.float32)
        + b1_ref[...]
    )
    hrelu = jnp.maximum(h_ref[...] + z, 0.0)
    o_ref[...] = (
        jnp.dot(hrelu, w2_ref[...], preferred_element_type=jnp.float32)
        + b2_ref[...]
    )


def _make_out(gsum, x_num, w1n, b1, w2p, b2p):
    return pl.pallas_call(
        _out_body,
        grid=(B // _OBLK,),
        in_specs=[
            pl.BlockSpec((_OBLK, H), lambda i: (i, 0)),
            pl.BlockSpec((_OBLK, NUM), lambda i: (i, 0)),
            pl.BlockSpec((NUM, H), lambda i: (0, 0)),
            pl.BlockSpec((1, H), lambda i: (0, 0)),
            pl.BlockSpec((H, _OPAD), lambda i: (0, 0)),
            pl.BlockSpec((1, _OPAD), lambda i: (0, 0)),
        ],
        out_specs=pl.BlockSpec((_OBLK, _OPAD), lambda i: (i, 0)),
        out_shape=jax.ShapeDtypeStruct((B, _OPAD), jnp.float32),
    )(gsum, x_num, w1n, b1.reshape(1, H), w2p, b2p)


def kernel(x_cat, x_num, tables, W1, b1, W2, b2):
    w1e = W1[: F * D].reshape(F, D, H)
    w1n = W1[F * D :]
    tw = _make_tw(tables, w1e)

    flat_idx = (x_cat.astype(jnp.int32) + jnp.arange(F, dtype=jnp.int32) * V).reshape(-1)
    gsum = _make_gsum(flat_idx, tw.reshape(F * V, H))

    w2p = jnp.zeros((H, _OPAD), jnp.float32).at[:, :OUT].set(W2)
    b2p = jnp.zeros((1, _OPAD), jnp.float32).at[0, :OUT].set(b2)
    out = _make_out(gsum, x_num, w1n, b1, w2p, b2p)
    return out[:, :OUT]


# speedup vs baseline: 44.6720x; 1.2738x over previous
import functools

import jax
import jax.numpy as jnp
from jax import lax
from jax.experimental import pallas as pl
from jax.experimental.pallas import tpu as pltpu
from jax.experimental.pallas import tpu_sc as plsc

B, F, V, D, NUM, H, OUT = 16384, 26, 1000, 50, 64, 128, 2

def _tw_body(t_ref, w_ref, tw_ref):
    tw_ref[0] = jnp.dot(t_ref[0], w_ref[0], preferred_element_type=jnp.float32)


def _make_tw(tables, w1e):
    return pl.pallas_call(
        _tw_body,
        grid=(F,),
        in_specs=[
            pl.BlockSpec((1, V, D), lambda f: (f, 0, 0)),
            pl.BlockSpec((1, D, H), lambda f: (f, 0, 0)),
        ],
        out_specs=pl.BlockSpec((1, V, H), lambda f: (f, 0, 0)),
        out_shape=jax.ShapeDtypeStruct((F, V, H), jnp.float32),
    )(tables, w1e)


NC, NS = 2, 16
NW = NC * NS
BPW = B // NW
C = 16
IPC = C * F
CHUNKS = BPW // C
_SPLITS = (128, 128, 128, 32)


def _sc_body(idx_hbm, tw_hbm, out_hbm, idx_v, rows0, rows1, acc0, acc1,
             gsem0, gsem1, osem0, osem1):
    wid = lax.axis_index("s") * NC + lax.axis_index("c")
    base_b = wid * BPW
    pltpu.sync_copy(idx_hbm.at[pl.ds(base_b * F, BPW * F)], idx_v)

    rows = (rows0, rows1)
    acc = (acc0, acc1)
    gsem = (gsem0, gsem1)
    osem = (osem0, osem1)

    def fire(ci, buf):
        off = 0
        for n in _SPLITS:
            pltpu.async_copy(
                tw_hbm.at[idx_v.at[pl.ds(ci * IPC + off, n)]],
                rows[buf].at[pl.ds(off, n)],
                gsem[buf],
            )
            off += n

    def drain_g(buf):
        pltpu.make_async_copy(tw_hbm.at[pl.ds(0, IPC)], rows[buf], gsem[buf]).wait()

    def drain_o(buf):
        pltpu.make_async_copy(acc[buf], out_hbm.at[pl.ds(0, C)], osem[buf]).wait()

    def compute(ci, buf):
        def b_body(bi, carry2):
            r0 = bi * F
            nh = H // 16
            accs = [rows[buf][r0, pl.ds(h * 16, 16)] for h in range(nh)]
            for f in range(1, F):
                for h in range(nh):
                    accs[h] = accs[h] + rows[buf][r0 + f, pl.ds(h * 16, 16)]
            for h in range(nh):
                acc[buf][bi, pl.ds(h * 16, 16)] = accs[h]
            return carry2

        lax.fori_loop(0, C, b_body, 0, unroll=False)
        pltpu.async_copy(acc[buf], out_hbm.at[pl.ds(base_b + ci * C, C)], osem[buf])

    fire(0, 0)
    K = CHUNKS // 2

    def body(k, carry):
        a = 2 * k
        fire(a + 1, 1)
        drain_g(0)

        @pl.when(k > 0)
        def _():
            drain_o(0)

        compute(a, 0)

        @pl.when(k < K - 1)
        def _():
            fire(a + 2, 0)

        drain_g(1)

        @pl.when(k > 0)
        def _():
            drain_o(1)

        compute(a + 1, 1)
        return carry

    lax.fori_loop(0, K, body, 0, unroll=False)
    drain_o(0)
    drain_o(1)


def _make_gsum(flat_idx, tw_flat):
    mesh = plsc.VectorSubcoreMesh(core_axis_name="c", subcore_axis_name="s")
    f = functools.partial(
        pl.kernel,
        _sc_body,
        mesh=mesh,
        out_type=jax.ShapeDtypeStruct((B, H), jnp.float32),
        scratch_types=[
            pltpu.VMEM((BPW * F,), jnp.int32),
            pltpu.VMEM((IPC, H), jnp.float32),
            pltpu.VMEM((IPC, H), jnp.float32),
            pltpu.VMEM((C, H), jnp.float32),
            pltpu.VMEM((C, H), jnp.float32),
            pltpu.SemaphoreType.DMA,
            pltpu.SemaphoreType.DMA,
            pltpu.SemaphoreType.DMA,
            pltpu.SemaphoreType.DMA,
        ],
    )()
    return f(flat_idx, tw_flat)


_OBLK = 2048
_OPAD = 8


def _out_body(h_ref, x_ref, w1n_ref, b1_ref, w2_ref, b2_ref, o_ref):
    z = (
        jnp.dot(x_ref[...], w1n_ref[...], preferred_element_type=jnp.float32)
        + b1_ref[...]
    )
    hrelu = jnp.maximum(h_ref[...] + z, 0.0)
    o_ref[...] = (
        jnp.dot(hrelu, w2_ref[...], preferred_element_type=jnp.float32)
        + b2_ref[...]
    )


def _make_out(gsum, x_num, w1n, b1, w2p, b2p):
    return pl.pallas_call(
        _out_body,
        grid=(B // _OBLK,),
        in_specs=[
            pl.BlockSpec((_OBLK, H), lambda i: (i, 0)),
            pl.BlockSpec((_OBLK, NUM), lambda i: (i, 0)),
            pl.BlockSpec((NUM, H), lambda i: (0, 0)),
            pl.BlockSpec((1, H), lambda i: (0, 0)),
            pl.BlockSpec((H, _OPAD), lambda i: (0, 0)),
            pl.BlockSpec((1, _OPAD), lambda i: (0, 0)),
        ],
        out_specs=pl.BlockSpec((_OBLK, _OPAD), lambda i: (i, 0)),
        out_shape=jax.ShapeDtypeStruct((B, _OPAD), jnp.float32),
    )(gsum, x_num, w1n, b1.reshape(1, H), w2p, b2p)


def kernel(x_cat, x_num, tables, W1, b1, W2, b2):
    w1e = W1[: F * D].reshape(F, D, H)
    w1n = W1[F * D :]
    tw = _make_tw(tables, w1e)

    flat_idx = (x_cat.astype(jnp.int32) + jnp.arange(F, dtype=jnp.int32) * V).reshape(-1)
    gsum = _make_gsum(flat_idx, tw.reshape(F * V, H))

    w2p = jnp.zeros((H, _OPAD), jnp.float32).at[:, :OUT].set(W2)
    b2p = jnp.zeros((1, _OPAD), jnp.float32).at[0, :OUT].set(b2)
    out = _make_out(gsum, x_num, w1n, b1, w2p, b2p)
    return out[:, :OUT]
